# baseline (speedup 1.0000x reference)
import functools

import jax
import jax.numpy as jnp
from jax import lax
from jax.experimental import pallas as pl
from jax.experimental.pallas import tpu as pltpu
from jax.experimental.pallas import tpu_sc as plsc

_N = 10000
_E = 160000
_G = 16
_D = 256
_H = 256
_HH = 128
_OUT = 10
_EPS = 1e-5

_R = 2000
_NB = _N // _R

_NC = 2
_NS = 16

_KE = 128
_EP = 163840
_CH = _EP // _KE
_CD = _CH // (_NC * _NS)
_CS = 81
_EPM = _CS * _NS * _KE

_RS = _N // _NS
_RI = 125
_RD = 624


def _sc_mesh():
    return plsc.VectorSubcoreMesh(core_axis_name="c", subcore_axis_name="s")


def _degree_body(dst_hbm, out_hbm, deg_sh, didx, ones_v, zb_v, sem):
    c = lax.axis_index("c")
    s = lax.axis_index("s")

    for j in range(_KE // 16):
        ones_v[pl.ds(j * 16, 16)] = jnp.ones((16,), jnp.float32)
    for j in range(640 // 16):
        zb_v[pl.ds(j * 16, 16)] = jnp.zeros((16,), jnp.float32)

    @pl.when(s < _NS - 1)
    def _():
        pltpu.sync_copy(zb_v.at[pl.ds(0, _RD)], deg_sh.at[pl.ds(s * _RD, _RD)])

    @pl.when(s == _NS - 1)
    def _():
        pltpu.sync_copy(zb_v, deg_sh.at[pl.ds((_NS - 1) * _RD, 640)])

    plsc.subcore_barrier()

    pltpu.sync_copy(dst_hbm.at[pl.ds(c * (_CH // _NC) + s * _CD, _CD)], didx)

    def grp(g, carry):
        for b in range(8):
            pltpu.async_copy(ones_v, deg_sh.at[didx.at[g * 8 + b]], sem,
                             add=True)
        for b in range(8):
            pltpu.make_async_copy(ones_v, deg_sh.at[didx.at[0]], sem).wait()
        return carry

    lax.fori_loop(0, _CD // 8, grp, 0)

    plsc.subcore_barrier()

    @pl.when(s < _NS - 1)
    def _():
        pltpu.sync_copy(deg_sh.at[pl.ds(s * _RD, _RD)], zb_v.at[pl.ds(0, _RD)])
        pltpu.sync_copy(zb_v.at[pl.ds(0, _RD)],
                        out_hbm.at[pl.ds(c * _N + s * _RD, _RD)])

    @pl.when(s == _NS - 1)
    def _():
        pltpu.sync_copy(deg_sh.at[pl.ds((_NS - 1) * _RD, 640)], zb_v)
        pltpu.sync_copy(zb_v,
                        out_hbm.at[pl.ds(c * _N + (_NS - 1) * _RD, 640)])


def _sc_degree(dst2):
    k = pl.kernel(
        _degree_body,
        out_type=jax.ShapeDtypeStruct((_NC * _N,), jnp.float32),
        mesh=_sc_mesh(),
        scratch_types=[
            pltpu.VMEM_SHARED((_N + 8,), jnp.float32),
            pltpu.VMEM((_CD, _KE), jnp.int32),
            pltpu.VMEM((_KE,), jnp.float32),
            pltpu.VMEM((640,), jnp.float32),
            pltpu.SemaphoreType.DMA,
        ],
        compiler_params=pltpu.CompilerParams(use_tc_tiling_on_sc=False),
    )
    return k(dst2)


def _scatter_body(hs_hbm, src_hbm, dst_hbm, out_hbm, acc_sh,
                  sb0, sb1, sb2, db0, db1, db2, b0, b1, b2,
                  e0, e1, e2, f0, f1, f2, g0, g1, g2):
    c = lax.axis_index("c")
    s = lax.axis_index("s")
    r0 = s * _RS
    base = s * _CS
    off = c * _N
    sbs = (sb0, sb1, sb2)
    dbs = (db0, db1, db2)
    bufs = (b0, b1, b2)
    es = (e0, e1, e2)
    fs = (f0, f1, f2)
    gs = (g0, g1, g2)

    def stage_src(l, S):
        pltpu.async_copy(src_hbm.at[pl.ds((base + l) * _KE, _KE)],
                         sbs[S], es[S])

    def wstage_src(S):
        pltpu.make_async_copy(src_hbm.at[pl.ds(0, _KE)],
                              sbs[S], es[S]).wait()

    def stage_dst(l, S):
        pltpu.async_copy(dst_hbm.at[pl.ds((base + l) * _KE, _KE)],
                         dbs[S], fs[S])

    def wstage_dst(S):
        pltpu.make_async_copy(dst_hbm.at[pl.ds(0, _KE)],
                              dbs[S], fs[S]).wait()

    def addoff(S):
        sb = sbs[S]
        for j in range(_KE // 16):
            sb[pl.ds(j * 16, 16)] = sb[pl.ds(j * 16, 16)] + off

    for S in range(3):
        stage_src(S, S)
        stage_dst(S, S)

    def iH(k, b, sm):
        pltpu.async_copy(hs_hbm.at[pl.ds(c * _N + r0 + k * _RI, _RI)],
                         b.at[pl.ds(0, _RI)], sm)

    def wiH(k, b, sm):
        pltpu.make_async_copy(hs_hbm.at[pl.ds(c * _N + r0 + k * _RI, _RI)],
                              b.at[pl.ds(0, _RI)], sm).wait()

    def iS(k, b, sm):
        pltpu.async_copy(b.at[pl.ds(0, _RI)],
                         acc_sh.at[pl.ds(r0 + k * _RI, _RI)], sm)

    def wiS(k, b, sm):
        pltpu.make_async_copy(b.at[pl.ds(0, _RI)],
                              acc_sh.at[pl.ds(r0 + k * _RI, _RI)], sm).wait()

    iH(0, b0, g0)
    iH(1, b1, g1)
    wiH(0, b0, g0)
    iS(0, b0, g0)
    wiH(1, b1, g1)
    iS(1, b1, g1)
    wiS(0, b0, g0)
    iH(2, b0, g0)
    wiH(2, b0, g0)
    iS(2, b0, g0)
    wiS(1, b1, g1)
    iH(3, b1, g1)
    wiH(3, b1, g1)
    iS(3, b1, g1)
    wiS(2, b0, g0)
    iH(4, b0, g0)
    wiH(4, b0, g0)
    iS(4, b0, g0)
    wiS(3, b1, g1)
    wiS(4, b0, g0)
    plsc.subcore_barrier()

    def gath(l, S):
        pltpu.async_copy(hs_hbm.at[sbs[S]], bufs[S], gs[S])

    def wgath(S):
        pltpu.make_async_copy(hs_hbm.at[sbs[S]], bufs[S], gs[S]).wait()

    def scat(l, S):
        pltpu.async_copy(bufs[S], acc_sh.at[dbs[S]], gs[S], add=True)

    def wscat(S):
        pltpu.make_async_copy(bufs[S], acc_sh.at[dbs[S]], gs[S]).wait()

    wstage_src(0)
    addoff(0)
    gath(0, 0)
    wstage_src(1)
    addoff(1)
    gath(1, 1)

    def chunk_steps(l, S):
        S2 = (S + 2) % 3

        @pl.when(l < _CS - 2)
        def _():
            wstage_src(S2)
            addoff(S2)

        @pl.when(l >= 1)
        def _():
            wscat(S2)

            @pl.when(l + 2 < _CS)
            def _():
                stage_dst(l + 2, S2)

        @pl.when(l < _CS - 2)
        def _():
            gath(l + 2, S2)

        wgath(S)
        wstage_dst(S)
        scat(l, S)

        @pl.when(l + 3 < _CS)
        def _():
            stage_src(l + 3, S)

    def body(i, carry):
        for j in range(3):
            chunk_steps(3 * i + j, j)
        return carry

    lax.fori_loop(0, _CS // 3, body, 0)
    wscat((_CS - 1) % 3)
    plsc.subcore_barrier()

    def oS(k, b, sm):
        pltpu.async_copy(acc_sh.at[pl.ds(r0 + k * _RI, _RI)],
                         b.at[pl.ds(0, _RI)], sm)

    def woS(k, b, sm):
        pltpu.make_async_copy(acc_sh.at[pl.ds(r0 + k * _RI, _RI)],
                              b.at[pl.ds(0, _RI)], sm).wait()

    def oH(k, b, sm):
        pltpu.async_copy(b.at[pl.ds(0, _RI)],
                         out_hbm.at[pl.ds(c * _N + r0 + k * _RI, _RI)], sm)

    def woH(k, b, sm):
        pltpu.make_async_copy(
            b.at[pl.ds(0, _RI)],
            out_hbm.at[pl.ds(c * _N + r0 + k * _RI, _RI)], sm).wait()

    oS(0, b0, g0)
    oS(1, b1, g1)
    woS(0, b0, g0)
    oH(0, b0, g0)
    woS(1, b1, g1)
    oH(1, b1, g1)
    woH(0, b0, g0)
    oS(2, b0, g0)
    woS(2, b0, g0)
    oH(2, b0, g0)
    woH(1, b1, g1)
    oS(3, b1, g1)
    woS(3, b1, g1)
    oH(3, b1, g1)
    woH(2, b0, g0)
    oS(4, b0, g0)
    woS(4, b0, g0)
    oH(4, b0, g0)
    woH(3, b1, g1)
    woH(4, b0, g0)


def _sc_scatter(hs, src1, dst2):
    k = pl.kernel(
        _scatter_body,
        out_type=jax.ShapeDtypeStruct((_NC * _N, _HH), jnp.float32),
        mesh=_sc_mesh(),
        scratch_types=(
            [pltpu.VMEM_SHARED((_N + 8, _HH), jnp.float32)]
            + [pltpu.VMEM((_KE,), jnp.int32) for _ in range(6)]
            + [pltpu.VMEM((_KE, _HH), jnp.float32) for _ in range(3)]
            + [pltpu.SemaphoreType.DMA for _ in range(9)]
        ),
        compiler_params=pltpu.CompilerParams(use_tc_tiling_on_sc=False),
    )
    return k(hs, src1, dst2)


def _mm1_body(deg0_ref, deg1_ref, h_ref, w_ref, hs_ref, dinv_ref):
    dinv = lax.rsqrt(deg0_ref[...] + deg1_ref[...] + 1.0)
    dinv_ref[...] = dinv
    mm = jnp.dot(h_ref[...], w_ref[...], preferred_element_type=jnp.float32)
    hs = dinv * mm
    hs_ref[0, :, :] = hs[:, 0:_HH]
    hs_ref[1, :, :] = hs[:, _HH:_H]


def _mm1(deg2, h, W):
    return pl.pallas_call(
        _mm1_body,
        grid=(_NB,),
        in_specs=[
            pl.BlockSpec((_R, 1), lambda i: (i, 0)),
            pl.BlockSpec((_R, 1), lambda i: (i + _NB, 0)),
            pl.BlockSpec((_R, _D), lambda i: (i, 0)),
            pl.BlockSpec((_D, _H), lambda i: (0, 0)),
        ],
        out_specs=[
            pl.BlockSpec((2, _R, _HH), lambda i: (0, i, 0)),
            pl.BlockSpec((_R, 1), lambda i: (i, 0)),
        ],
        out_shape=[
            jax.ShapeDtypeStruct((2, _N, _HH), jnp.float32),
            jax.ShapeDtypeStruct((_N, 1), jnp.float32),
        ],
    )(deg2, deg2, h, W)


def _stats_body(dinv_ref, acc_ref, st_ref, s1, s2):
    i = pl.program_id(0)
    conv = dinv_ref[...] * jnp.concatenate(
        [acc_ref[0, :, :], acc_ref[1, :, :]], axis=1)

    @pl.when(i == 0)
    def _():
        s1[...] = jnp.zeros_like(s1)
        s2[...] = jnp.zeros_like(s2)

    s1[...] += jnp.sum(conv, axis=0, keepdims=True)
    s2[...] += jnp.sum(conv * conv, axis=0, keepdims=True)

    @pl.when(i == _NB - 1)
    def _():
        st_ref[0:1, :] = s1[...]
        st_ref[1:2, :] = s2[...]


def _stats(dinv, acc):
    return pl.pallas_call(
        _stats_body,
        grid=(_NB,),
        in_specs=[
            pl.BlockSpec((_R, 1), lambda i: (i, 0)),
            pl.BlockSpec((2, _R, _HH), lambda i: (0, i, 0)),
        ],
        out_specs=pl.BlockSpec((2, _H), lambda i: (0, 0)),
        out_shape=jax.ShapeDtypeStruct((2, _H), jnp.float32),
        scratch_shapes=[
            pltpu.VMEM((1, _H), jnp.float32),
            pltpu.VMEM((1, _H), jnp.float32),
        ],
    )(dinv, acc)


def _bn_mm_body(st_ref, g_ref, be_ref, dinv_ref, acc_ref, w_ref, hs_ref):
    m = st_ref[0:1, :] / _N
    var = st_ref[1:2, :] / _N - m * m
    a = lax.rsqrt(var + _EPS) * g_ref[...]
    cc = be_ref[...] - m * a
    conv = dinv_ref[...] * jnp.concatenate(
        [acc_ref[0, :, :], acc_ref[1, :, :]], axis=1)
    hn = jnp.maximum(conv * a + cc, 0.0)
    hs = dinv_ref[...] * jnp.dot(hn, w_ref[...],
                                 preferred_element_type=jnp.float32)
    hs_ref[0, :, :] = hs[:, 0:_HH]
    hs_ref[1, :, :] = hs[:, _HH:_H]


def _bn_mm(st, g, be, dinv, acc, W):
    return pl.pallas_call(
        _bn_mm_body,
        grid=(_NB,),
        in_specs=[
            pl.BlockSpec((2, _H), lambda i: (0, 0)),
            pl.BlockSpec((1, _H), lambda i: (0, 0)),
            pl.BlockSpec((1, _H), lambda i: (0, 0)),
            pl.BlockSpec((_R, 1), lambda i: (i, 0)),
            pl.BlockSpec((2, _R, _HH), lambda i: (0, i, 0)),
            pl.BlockSpec((_H, _H), lambda i: (0, 0)),
        ],
        out_specs=pl.BlockSpec((2, _R, _HH), lambda i: (0, i, 0)),
        out_shape=jax.ShapeDtypeStruct((2, _N, _HH), jnp.float32),
    )(st, g[None, :], be[None, :], dinv, acc, W)


def _pool_cls_body(st_ref, g_ref, be_ref, batch_ref, dinv_ref, acc_ref,
                   wc1_ref, bc1_ref, wc2_ref, bc2_ref, wc3_ref, bc3_ref,
                   out_ref, s_acc, mx_acc, cnt_acc):
    i = pl.program_id(0)
    m = st_ref[0:1, :] / _N
    var = st_ref[1:2, :] / _N - m * m
    a = lax.rsqrt(var + _EPS) * g_ref[...]
    cc = be_ref[...] - m * a
    conv = dinv_ref[...] * jnp.concatenate(
        [acc_ref[0, :, :], acc_ref[1, :, :]], axis=1)
    hn = jnp.maximum(conv * a + cc, 0.0)
    b = batch_ref[...]
    gids = jax.lax.broadcasted_iota(jnp.int32, (1, _G), 1).astype(jnp.float32)
    onehot = (b == gids).astype(jnp.float32)

    @pl.when(i == 0)
    def _():
        s_acc[...] = jnp.zeros_like(s_acc)
        cnt_acc[...] = jnp.zeros_like(cnt_acc)
        mx_acc[...] = jnp.full_like(mx_acc, -jnp.inf)

    s_acc[...] += jax.lax.dot_general(
        onehot, hn, (((0,), (0,)), ((), ())),
        preferred_element_type=jnp.float32)
    cnt_acc[...] += jax.lax.dot_general(
        onehot, jnp.ones((_R, 1), jnp.float32), (((0,), (0,)), ((), ())),
        preferred_element_type=jnp.float32)
    for gi in range(_G):
        mg = jnp.max(jnp.where(b == float(gi), hn, -jnp.inf),
                     axis=0, keepdims=True)
        mx_acc[gi:gi + 1, :] = jnp.maximum(mx_acc[gi:gi + 1, :], mg)

    @pl.when(i == _NB - 1)
    def _():
        sg = s_acc[...]
        mean = sg / jnp.maximum(cnt_acc[...], 1.0)
        mx = mx_acc[...]
        z1 = jnp.maximum(
            jnp.dot(mean, wc1_ref[0:_H, :], preferred_element_type=jnp.float32)
            + jnp.dot(mx, wc1_ref[_H:2 * _H, :], preferred_element_type=jnp.float32)
            + jnp.dot(sg, wc1_ref[2 * _H:3 * _H, :], preferred_element_type=jnp.float32)
            + bc1_ref[...], 0.0)
        z2 = jnp.maximum(
            jnp.dot(z1, wc2_ref[...], preferred_element_type=jnp.float32)
            + bc2_ref[...], 0.0)
        out_ref[...] = jnp.dot(
            z2, wc3_ref[...], preferred_element_type=jnp.float32) + bc3_ref[...]


def _pool_cls(st, g, be, batch_f, dinv, acc, Wc1, bc1, Wc2, bc2, Wc3, bc3):
    return pl.pallas_call(
        _pool_cls_body,
        grid=(_NB,),
        in_specs=[
            pl.BlockSpec((2, _H), lambda i: (0, 0)),
            pl.BlockSpec((1, _H), lambda i: (0, 0)),
            pl.BlockSpec((1, _H), lambda i: (0, 0)),
            pl.BlockSpec((_R, 1), lambda i: (i, 0)),
            pl.BlockSpec((_R, 1), lambda i: (i, 0)),
            pl.BlockSpec((2, _R, _HH), lambda i: (0, i, 0)),
            pl.BlockSpec((3 * _H, 2 * _H), lambda i: (0, 0)),
            pl.BlockSpec((1, 2 * _H), lambda i: (0, 0)),
            pl.BlockSpec((2 * _H, _H), lambda i: (0, 0)),
            pl.BlockSpec((1, _H), lambda i: (0, 0)),
            pl.BlockSpec((_H, _OUT), lambda i: (0, 0)),
            pl.BlockSpec((1, _OUT), lambda i: (0, 0)),
        ],
        out_specs=pl.BlockSpec((_G, _OUT), lambda i: (0, 0)),
        out_shape=jax.ShapeDtypeStruct((_G, _OUT), jnp.float32),
        scratch_shapes=[
            pltpu.VMEM((_G, _H), jnp.float32),
            pltpu.VMEM((_G, _H), jnp.float32),
            pltpu.VMEM((_G, 1), jnp.float32),
        ],
    )(st, g[None, :], be[None, :], batch_f, dinv, acc,
      Wc1, bc1[None, :], Wc2, bc2[None, :], Wc3, bc3[None, :])


def kernel(x, edge_index, batch, W1, b1, W2, b2, W3, b3,
           g1, be1, g2, be2, g3, be3, Wc1, bc1, Wc2, bc2, Wc3, bc3):
    src = edge_index[0]
    dst = edge_index[1]
    padm = _EPM - _E
    src1 = jnp.concatenate([src, jnp.zeros((padm,), jnp.int32)])
    dst1 = jnp.concatenate([dst, jnp.full((padm,), _N, jnp.int32)])
    dst2 = jnp.concatenate(
        [dst, jnp.full((_EP - _E,), _N, jnp.int32)]).reshape(_CH, _KE)
    batch_f = batch.astype(jnp.float32)[:, None]

    deg2 = _sc_degree(dst2).reshape(_NC * _N, 1)
    hs3, dinv = _mm1(deg2, x, W1)

    for (W_next, g, be) in ((W2, g1, be1), (W3, g2, be2)):
        acc = _sc_scatter(hs3.reshape(_NC * _N, _HH), src1, dst1)
        acc3 = acc.reshape(_NC, _N, _HH)
        st = _stats(dinv, acc3)
        hs3 = _bn_mm(st, g, be, dinv, acc3, W_next)

    acc = _sc_scatter(hs3.reshape(_NC * _N, _HH), src1, dst1)
    acc3 = acc.reshape(_NC, _N, _HH)
    st = _stats(dinv, acc3)
    return _pool_cls(st, g3, be3, batch_f, dinv, acc3,
                     Wc1, bc1, Wc2, bc2, Wc3, bc3)


# speedup vs baseline: 1.3971x; 1.3971x over previous
import functools

import jax
import jax.numpy as jnp
from jax import lax
from jax.experimental import pallas as pl
from jax.experimental.pallas import tpu as pltpu
from jax.experimental.pallas import tpu_sc as plsc

_N = 10000
_E = 160000
_G = 16
_D = 256
_H = 256
_HH = 128
_OUT = 10
_EPS = 1e-5

_R = 2000
_NB = _N // _R

_NC = 2
_NS = 16

_KE = 128
_EP = 163840
_CH = _EP // _KE
_CS = _CH // _NS
_CD = _CH // (_NC * _NS)

_RS = _N // _NS
_RI = 125
_RD = 624


def _sc_mesh():
    return plsc.VectorSubcoreMesh(core_axis_name="c", subcore_axis_name="s")


def _degree_body(dst_hbm, out_hbm, deg_sh, didx, ones_v, zb_v, sem):
    c = lax.axis_index("c")
    s = lax.axis_index("s")

    for j in range(_KE // 16):
        ones_v[pl.ds(j * 16, 16)] = jnp.ones((16,), jnp.float32)
    for j in range(640 // 16):
        zb_v[pl.ds(j * 16, 16)] = jnp.zeros((16,), jnp.float32)

    @pl.when(s < _NS - 1)
    def _():
        pltpu.sync_copy(zb_v.at[pl.ds(0, _RD)], deg_sh.at[pl.ds(s * _RD, _RD)])

    @pl.when(s == _NS - 1)
    def _():
        pltpu.sync_copy(zb_v, deg_sh.at[pl.ds((_NS - 1) * _RD, 640)])

    plsc.subcore_barrier()

    pltpu.sync_copy(dst_hbm.at[pl.ds(c * (_CH // _NC) + s * _CD, _CD)], didx)

    def grp(g, carry):
        for b in range(8):
            pltpu.async_copy(ones_v, deg_sh.at[didx.at[g * 8 + b]], sem,
                             add=True)
        for b in range(8):
            pltpu.make_async_copy(ones_v, deg_sh.at[didx.at[0]], sem).wait()
        return carry

    lax.fori_loop(0, _CD // 8, grp, 0)

    plsc.subcore_barrier()

    @pl.when(s < _NS - 1)
    def _():
        pltpu.sync_copy(deg_sh.at[pl.ds(s * _RD, _RD)], zb_v.at[pl.ds(0, _RD)])
        pltpu.sync_copy(zb_v.at[pl.ds(0, _RD)],
                        out_hbm.at[pl.ds(c * _N + s * _RD, _RD)])

    @pl.when(s == _NS - 1)
    def _():
        pltpu.sync_copy(deg_sh.at[pl.ds((_NS - 1) * _RD, 640)], zb_v)
        pltpu.sync_copy(zb_v,
                        out_hbm.at[pl.ds(c * _N + (_NS - 1) * _RD, 640)])


def _sc_degree(dst2):
    k = pl.kernel(
        _degree_body,
        out_type=jax.ShapeDtypeStruct((_NC * _N,), jnp.float32),
        mesh=_sc_mesh(),
        scratch_types=[
            pltpu.VMEM_SHARED((_N + 8,), jnp.float32),
            pltpu.VMEM((_CD, _KE), jnp.int32),
            pltpu.VMEM((_KE,), jnp.float32),
            pltpu.VMEM((640,), jnp.float32),
            pltpu.SemaphoreType.DMA,
        ],
        compiler_params=pltpu.CompilerParams(use_tc_tiling_on_sc=False),
    )
    return k(dst2)


def _scatter_body(hs_hbm, src_hbm, dst_hbm, out_hbm, acc_sh,
                  didx, sb0, sb1, b0, b1, e0, e1, g0, g1):
    c = lax.axis_index("c")
    s = lax.axis_index("s")
    r0 = s * _RS
    base = s * _CS
    off = c * _N
    sbs = (sb0, sb1)
    ses = (e0, e1)
    bufs = (b0, b1)
    gs = (g0, g1)

    pltpu.sync_copy(dst_hbm.at[pl.ds(base, _CS)], didx)

    def stage(l, sb, sm):
        pltpu.async_copy(src_hbm.at[pl.ds((base + l) * _KE, _KE)], sb, sm)

    def wstage(sb, sm):
        pltpu.make_async_copy(src_hbm.at[pl.ds(0, _KE)], sb, sm).wait()

    def addoff(sb):
        for j in range(_KE // 16):
            sb[pl.ds(j * 16, 16)] = sb[pl.ds(j * 16, 16)] + off

    stage(0, sb0, e0)
    stage(1, sb1, e1)

    def iH(k, b, sm):
        pltpu.async_copy(hs_hbm.at[pl.ds(c * _N + r0 + k * _RI, _RI)],
                         b.at[pl.ds(0, _RI)], sm)

    def wiH(k, b, sm):
        pltpu.make_async_copy(hs_hbm.at[pl.ds(c * _N + r0 + k * _RI, _RI)],
                              b.at[pl.ds(0, _RI)], sm).wait()

    def iS(k, b, sm):
        pltpu.async_copy(b.at[pl.ds(0, _RI)],
                         acc_sh.at[pl.ds(r0 + k * _RI, _RI)], sm)

    def wiS(k, b, sm):
        pltpu.make_async_copy(b.at[pl.ds(0, _RI)],
                              acc_sh.at[pl.ds(r0 + k * _RI, _RI)], sm).wait()

    iH(0, b0, g0)
    iH(1, b1, g1)
    wiH(0, b0, g0)
    iS(0, b0, g0)
    wiH(1, b1, g1)
    iS(1, b1, g1)
    wiS(0, b0, g0)
    iH(2, b0, g0)
    wiH(2, b0, g0)
    iS(2, b0, g0)
    wiS(1, b1, g1)
    iH(3, b1, g1)
    wiH(3, b1, g1)
    iS(3, b1, g1)
    wiS(2, b0, g0)
    iH(4, b0, g0)
    wiH(4, b0, g0)
    iS(4, b0, g0)
    wiS(3, b1, g1)
    wiS(4, b0, g0)
    plsc.subcore_barrier()

    def wgath(b, sm):
        pltpu.make_async_copy(hs_hbm.at[sb0], b, sm).wait()

    def scat(l, b, sm):
        pltpu.async_copy(b, acc_sh.at[didx.at[l]], sm, add=True)

    def wscat(b, sm):
        pltpu.make_async_copy(b, acc_sh.at[didx.at[0]], sm).wait()

    wstage(sb0, e0)
    addoff(sb0)
    pltpu.async_copy(hs_hbm.at[sb0], b0, g0)

    def body(l, carry):
        b = l % 2

        def run(sb_b, e_b, buf_b, g_b, sb_n, e_n, buf_n, g_n):
            @pl.when(l < _CS - 1)
            def _():
                wstage(sb_n, e_n)
                addoff(sb_n)

                @pl.when(l >= 1)
                def _():
                    wscat(buf_n, g_n)

                pltpu.async_copy(hs_hbm.at[sb_n], buf_n, g_n)

            wgath(buf_b, g_b)

            @pl.when(l < _CS - 2)
            def _():
                stage(l + 2, sb_b, e_b)

            scat(l, buf_b, g_b)

        @pl.when(b == 0)
        def _():
            run(sb0, e0, b0, g0, sb1, e1, b1, g1)

        @pl.when(b == 1)
        def _():
            run(sb1, e1, b1, g1, sb0, e0, b0, g0)

        return carry

    lax.fori_loop(0, _CS, body, 0)
    wscat(b0, g0)
    wscat(b1, g1)
    plsc.subcore_barrier()

    def oS(k, b, sm):
        pltpu.async_copy(acc_sh.at[pl.ds(r0 + k * _RI, _RI)],
                         b.at[pl.ds(0, _RI)], sm)

    def woS(k, b, sm):
        pltpu.make_async_copy(acc_sh.at[pl.ds(r0 + k * _RI, _RI)],
                              b.at[pl.ds(0, _RI)], sm).wait()

    def oH(k, b, sm):
        pltpu.async_copy(b.at[pl.ds(0, _RI)],
                         out_hbm.at[pl.ds(c * _N + r0 + k * _RI, _RI)], sm)

    def woH(k, b, sm):
        pltpu.make_async_copy(
            b.at[pl.ds(0, _RI)],
            out_hbm.at[pl.ds(c * _N + r0 + k * _RI, _RI)], sm).wait()

    oS(0, b0, g0)
    oS(1, b1, g1)
    woS(0, b0, g0)
    oH(0, b0, g0)
    woS(1, b1, g1)
    oH(1, b1, g1)
    woH(0, b0, g0)
    oS(2, b0, g0)
    woS(2, b0, g0)
    oH(2, b0, g0)
    woH(1, b1, g1)
    oS(3, b1, g1)
    woS(3, b1, g1)
    oH(3, b1, g1)
    woH(2, b0, g0)
    oS(4, b0, g0)
    woS(4, b0, g0)
    oH(4, b0, g0)
    woH(3, b1, g1)
    woH(4, b0, g0)


def _sc_scatter(hs, src1, dst2):
    k = pl.kernel(
        _scatter_body,
        out_type=jax.ShapeDtypeStruct((_NC * _N, _HH), jnp.float32),
        mesh=_sc_mesh(),
        scratch_types=[
            pltpu.VMEM_SHARED((_N + 8, _HH), jnp.float32),
            pltpu.VMEM((_CS, _KE), jnp.int32),
            pltpu.VMEM((_KE,), jnp.int32),
            pltpu.VMEM((_KE,), jnp.int32),
            pltpu.VMEM((_KE, _HH), jnp.float32),
            pltpu.VMEM((_KE, _HH), jnp.float32),
            pltpu.SemaphoreType.DMA,
            pltpu.SemaphoreType.DMA,
            pltpu.SemaphoreType.DMA,
            pltpu.SemaphoreType.DMA,
        ],
        compiler_params=pltpu.CompilerParams(use_tc_tiling_on_sc=False),
    )
    return k(hs, src1, dst2)


def _mm1_body(deg0_ref, deg1_ref, h_ref, w_ref, hs_ref, dinv_ref):
    dinv = lax.rsqrt(deg0_ref[...] + deg1_ref[...] + 1.0)
    dinv_ref[...] = dinv
    mm = jnp.dot(h_ref[...], w_ref[...], preferred_element_type=jnp.float32)
    hs = dinv * mm
    hs_ref[0, :, :] = hs[:, 0:_HH]
    hs_ref[1, :, :] = hs[:, _HH:_H]


def _mm1(deg2, h, W):
    return pl.pallas_call(
        _mm1_body,
        grid=(_NB,),
        in_specs=[
            pl.BlockSpec((_R, 1), lambda i: (i, 0)),
            pl.BlockSpec((_R, 1), lambda i: (i + _NB, 0)),
            pl.BlockSpec((_R, _D), lambda i: (i, 0)),
            pl.BlockSpec((_D, _H), lambda i: (0, 0)),
        ],
        out_specs=[
            pl.BlockSpec((2, _R, _HH), lambda i: (0, i, 0)),
            pl.BlockSpec((_R, 1), lambda i: (i, 0)),
        ],
        out_shape=[
            jax.ShapeDtypeStruct((2, _N, _HH), jnp.float32),
            jax.ShapeDtypeStruct((_N, 1), jnp.float32),
        ],
    )(deg2, deg2, h, W)


def _stats_body(dinv_ref, acc_ref, st_ref, s1, s2):
    i = pl.program_id(0)
    conv = dinv_ref[...] * jnp.concatenate(
        [acc_ref[0, :, :], acc_ref[1, :, :]], axis=1)

    @pl.when(i == 0)
    def _():
        s1[...] = jnp.zeros_like(s1)
        s2[...] = jnp.zeros_like(s2)

    s1[...] += jnp.sum(conv, axis=0, keepdims=True)
    s2[...] += jnp.sum(conv * conv, axis=0, keepdims=True)

    @pl.when(i == _NB - 1)
    def _():
        st_ref[0:1, :] = s1[...]
        st_ref[1:2, :] = s2[...]


def _stats(dinv, acc):
    return pl.pallas_call(
        _stats_body,
        grid=(_NB,),
        in_specs=[
            pl.BlockSpec((_R, 1), lambda i: (i, 0)),
            pl.BlockSpec((2, _R, _HH), lambda i: (0, i, 0)),
        ],
        out_specs=pl.BlockSpec((2, _H), lambda i: (0, 0)),
        out_shape=jax.ShapeDtypeStruct((2, _H), jnp.float32),
        scratch_shapes=[
            pltpu.VMEM((1, _H), jnp.float32),
            pltpu.VMEM((1, _H), jnp.float32),
        ],
    )(dinv, acc)


def _bn_mm_body(st_ref, g_ref, be_ref, dinv_ref, acc_ref, w_ref, hs_ref):
    m = st_ref[0:1, :] / _N
    var = st_ref[1:2, :] / _N - m * m
    a = lax.rsqrt(var + _EPS) * g_ref[...]
    cc = be_ref[...] - m * a
    conv = dinv_ref[...] * jnp.concatenate(
        [acc_ref[0, :, :], acc_ref[1, :, :]], axis=1)
    hn = jnp.maximum(conv * a + cc, 0.0)
    hs = dinv_ref[...] * jnp.dot(hn, w_ref[...],
                                 preferred_element_type=jnp.float32)
    hs_ref[0, :, :] = hs[:, 0:_HH]
    hs_ref[1, :, :] = hs[:, _HH:_H]


def _bn_mm(st, g, be, dinv, acc, W):
    return pl.pallas_call(
        _bn_mm_body,
        grid=(_NB,),
        in_specs=[
            pl.BlockSpec((2, _H), lambda i: (0, 0)),
            pl.BlockSpec((1, _H), lambda i: (0, 0)),
            pl.BlockSpec((1, _H), lambda i: (0, 0)),
            pl.BlockSpec((_R, 1), lambda i: (i, 0)),
            pl.BlockSpec((2, _R, _HH), lambda i: (0, i, 0)),
            pl.BlockSpec((_H, _H), lambda i: (0, 0)),
        ],
        out_specs=pl.BlockSpec((2, _R, _HH), lambda i: (0, i, 0)),
        out_shape=jax.ShapeDtypeStruct((2, _N, _HH), jnp.float32),
    )(st, g[None, :], be[None, :], dinv, acc, W)


def _pool_cls_body(st_ref, g_ref, be_ref, batch_ref, dinv_ref, acc_ref,
                   wc1_ref, bc1_ref, wc2_ref, bc2_ref, wc3_ref, bc3_ref,
                   out_ref, s_acc, mx_acc, cnt_acc):
    i = pl.program_id(0)
    m = st_ref[0:1, :] / _N
    var = st_ref[1:2, :] / _N - m * m
    a = lax.rsqrt(var + _EPS) * g_ref[...]
    cc = be_ref[...] - m * a
    conv = dinv_ref[...] * jnp.concatenate(
        [acc_ref[0, :, :], acc_ref[1, :, :]], axis=1)
    hn = jnp.maximum(conv * a + cc, 0.0)
    b = batch_ref[...]
    gids = jax.lax.broadcasted_iota(jnp.int32, (1, _G), 1).astype(jnp.float32)
    onehot = (b == gids).astype(jnp.float32)

    @pl.when(i == 0)
    def _():
        s_acc[...] = jnp.zeros_like(s_acc)
        cnt_acc[...] = jnp.zeros_like(cnt_acc)
        mx_acc[...] = jnp.full_like(mx_acc, -jnp.inf)

    s_acc[...] += jax.lax.dot_general(
        onehot, hn, (((0,), (0,)), ((), ())),
        preferred_element_type=jnp.float32)
    cnt_acc[...] += jax.lax.dot_general(
        onehot, jnp.ones((_R, 1), jnp.float32), (((0,), (0,)), ((), ())),
        preferred_element_type=jnp.float32)
    for gi in range(_G):
        mg = jnp.max(jnp.where(b == float(gi), hn, -jnp.inf),
                     axis=0, keepdims=True)
        mx_acc[gi:gi + 1, :] = jnp.maximum(mx_acc[gi:gi + 1, :], mg)

    @pl.when(i == _NB - 1)
    def _():
        sg = s_acc[...]
        mean = sg / jnp.maximum(cnt_acc[...], 1.0)
        mx = mx_acc[...]
        z1 = jnp.maximum(
            jnp.dot(mean, wc1_ref[0:_H, :], preferred_element_type=jnp.float32)
            + jnp.dot(mx, wc1_ref[_H:2 * _H, :], preferred_element_type=jnp.float32)
            + jnp.dot(sg, wc1_ref[2 * _H:3 * _H, :], preferred_element_type=jnp.float32)
            + bc1_ref[...], 0.0)
        z2 = jnp.maximum(
            jnp.dot(z1, wc2_ref[...], preferred_element_type=jnp.float32)
            + bc2_ref[...], 0.0)
        out_ref[...] = jnp.dot(
            z2, wc3_ref[...], preferred_element_type=jnp.float32) + bc3_ref[...]


def _pool_cls(st, g, be, batch_f, dinv, acc, Wc1, bc1, Wc2, bc2, Wc3, bc3):
    return pl.pallas_call(
        _pool_cls_body,
        grid=(_NB,),
        in_specs=[
            pl.BlockSpec((2, _H), lambda i: (0, 0)),
            pl.BlockSpec((1, _H), lambda i: (0, 0)),
            pl.BlockSpec((1, _H), lambda i: (0, 0)),
            pl.BlockSpec((_R, 1), lambda i: (i, 0)),
            pl.BlockSpec((_R, 1), lambda i: (i, 0)),
            pl.BlockSpec((2, _R, _HH), lambda i: (0, i, 0)),
            pl.BlockSpec((3 * _H, 2 * _H), lambda i: (0, 0)),
            pl.BlockSpec((1, 2 * _H), lambda i: (0, 0)),
            pl.BlockSpec((2 * _H, _H), lambda i: (0, 0)),
            pl.BlockSpec((1, _H), lambda i: (0, 0)),
            pl.BlockSpec((_H, _OUT), lambda i: (0, 0)),
            pl.BlockSpec((1, _OUT), lambda i: (0, 0)),
        ],
        out_specs=pl.BlockSpec((_G, _OUT), lambda i: (0, 0)),
        out_shape=jax.ShapeDtypeStruct((_G, _OUT), jnp.float32),
        scratch_shapes=[
            pltpu.VMEM((_G, _H), jnp.float32),
            pltpu.VMEM((_G, _H), jnp.float32),
            pltpu.VMEM((_G, 1), jnp.float32),
        ],
    )(st, g[None, :], be[None, :], batch_f, dinv, acc,
      Wc1, bc1[None, :], Wc2, bc2[None, :], Wc3, bc3[None, :])


def kernel(x, edge_index, batch, W1, b1, W2, b2, W3, b3,
           g1, be1, g2, be2, g3, be3, Wc1, bc1, Wc2, bc2, Wc3, bc3):
    src = edge_index[0]
    dst = edge_index[1]
    pad = _EP - _E
    src1 = jnp.concatenate([src, jnp.zeros((pad,), jnp.int32)])
    dst2 = jnp.concatenate(
        [dst, jnp.full((pad,), _N, jnp.int32)]).reshape(_CH, _KE)
    batch_f = batch.astype(jnp.float32)[:, None]

    deg2 = _sc_degree(dst2).reshape(_NC * _N, 1)
    hs3, dinv = _mm1(deg2, x, W1)

    for (W_next, g, be) in ((W2, g1, be1), (W3, g2, be2)):
        acc = _sc_scatter(hs3.reshape(_NC * _N, _HH), src1, dst2)
        acc3 = acc.reshape(_NC, _N, _HH)
        st = _stats(dinv, acc3)
        hs3 = _bn_mm(st, g, be, dinv, acc3, W_next)

    acc = _sc_scatter(hs3.reshape(_NC * _N, _HH), src1, dst2)
    acc3 = acc.reshape(_NC, _N, _HH)
    st = _stats(dinv, acc3)
    return _pool_cls(st, g3, be3, batch_f, dinv, acc3,
                     Wc1, bc1, Wc2, bc2, Wc3, bc3)
